# baseline (speedup 1.0000x reference)
import functools

import jax
import jax.numpy as jnp
from jax import lax
from jax.experimental import pallas as pl
from jax.experimental.pallas import tpu as pltpu
from jax.experimental.pallas import tpu_sc as plsc

N = 10000
E = 320000
D_IN = 128
D_H = 256
D_OUT = 128

NC = 2
NS = 16
LANES = 16

N_PAD = 10240
ROW_SLICE = N_PAD // NS
E_TOT = E + N
K = 128
E_PAD = 8192 * 41
E_F = 4096 * 81
PT_L = E_F // NS
PT_D = E_PAD // (NC * NS)

@functools.cache
def _mesh():
    return plsc.VectorSubcoreMesh(
        core_axis_name="c", subcore_axis_name="s", num_cores=NC, num_subcores=NS
    )


def _zero_1d(buf, n):
    def w(i, _):
        buf[pl.ds(i * LANES, LANES)] = jnp.zeros((LANES,), jnp.float32)
        return 0
    lax.fori_loop(0, n // LANES, w, 0)


def _zero_2d(buf, rows, cols):
    def w(r, _):
        for j in range(cols // LANES):
            buf[r, pl.ds(j * LANES, LANES)] = jnp.zeros((LANES,), jnp.float32)
        return 0
    lax.fori_loop(0, rows, w, 0)


@functools.cache
def _get_deg_kernel():
    return pl.kernel(
        _deg_body,
        out_type=jax.ShapeDtypeStruct((NC, N_PAD), jnp.float32),
        mesh=_mesh(),
        scratch_types=[
            pltpu.VMEM((K,), jnp.int32),
            pltpu.VMEM((K,), jnp.int32),
            pltpu.VMEM((K,), jnp.float32),
            pltpu.VMEM((ROW_SLICE,), jnp.float32),
            pltpu.VMEM_SHARED((N_PAD,), jnp.float32),
            pltpu.SemaphoreType.DMA,
            pltpu.SemaphoreType.DMA,
        ],
    )


def _deg_body(dst_hbm, out_hbm, d0, d1, ones_v, buf_v, acc, isem0, isem1):
    c = lax.axis_index("c")
    s = lax.axis_index("s")
    idxs = (d0, d1)
    isems = (isem0, isem1)

    def w1(i, _):
        ones_v[pl.ds(i * LANES, LANES)] = jnp.full((LANES,), 1.0, jnp.float32)
        return 0
    lax.fori_loop(0, K // LANES, w1, 0)
    _zero_1d(buf_v, ROW_SLICE)
    pltpu.sync_copy(buf_v, acc.at[pl.ds(s * ROW_SLICE, ROW_SLICE)])

    base = c * (E_PAD // NC) + s * PT_D
    nb = (PT_D // K) // 2

    def issue_idx(g, p):
        pltpu.async_copy(dst_hbm.at[pl.ds(base + g * K, K)],
                         idxs[p], isems[p])

    def wait_idx(p):
        pltpu.make_async_copy(dst_hbm.at[pl.ds(0, K)],
                              idxs[p], isems[p]).wait()

    issue_idx(0, 0)
    plsc.subcore_barrier()

    def body(t, _):
        wait_idx(0)
        issue_idx(2 * t + 1, 1)
        pltpu.sync_copy(ones_v, acc.at[d0], add=True)
        wait_idx(1)

        @pl.when(t + 1 < nb)
        def _():
            issue_idx(2 * (t + 1), 0)
        pltpu.sync_copy(ones_v, acc.at[d1], add=True)
        return 0
    lax.fori_loop(0, nb, body, 0)

    plsc.subcore_barrier()
    pltpu.sync_copy(acc.at[pl.ds(s * ROW_SLICE, ROW_SLICE)], buf_v)
    pltpu.sync_copy(buf_v, out_hbm.at[c, pl.ds(s * ROW_SLICE, ROW_SLICE)])


@functools.cache
def _make_edge_kernel(mode):
    feat = mode == "feat"
    dh = 128
    nch = (PT_L if feat else PT_D) // K
    nb = nch // 2
    assert nb * 2 == nch

    def edge_kernel(table_hbm, srcb_hbm, dst_hbm, out_hbm,
                    src0, src1, dst0, dst1, rows0, rows1, acc,
                    gsem0, gsem1, isem0, isem1):
        c = lax.axis_index("c")
        s = lax.axis_index("s")
        srcs = (src0, src1)
        dsts = (dst0, dst1)
        rows = (rows0, rows1)
        gsems = (gsem0, gsem1)
        isems = (isem0, isem1)

        if feat:
            sbase = c * E_PAD + s * PT_L
            dbase = s * PT_L
        else:
            sbase = c * (E_PAD // NC) + s * PT_D
            dbase = sbase

        def issue_idx(g, p):
            pltpu.async_copy(srcb_hbm.at[pl.ds(sbase + g * K, K)],
                             srcs[p], isems[p])
            pltpu.async_copy(dst_hbm.at[pl.ds(dbase + g * K, K)],
                             dsts[p], isems[p])

        def wait_idx(p):
            pltpu.make_async_copy(srcb_hbm.at[pl.ds(0, K)],
                                  srcs[p], isems[p]).wait()
            pltpu.make_async_copy(dst_hbm.at[pl.ds(0, K)],
                                  dsts[p], isems[p]).wait()

        def issue_gather(p):
            pltpu.async_copy(table_hbm.at[srcs[p]], rows[p], gsems[p])

        def wait_gather(p):
            pltpu.make_async_copy(table_hbm.at[pl.ds(0, K)],
                                  rows[p], gsems[p]).wait()

        def sync_scatter(p):
            pltpu.sync_copy(rows[p], acc.at[dsts[p]], add=True)

        _zero_2d(rows0, K, dh)
        for r in range(ROW_SLICE // K):
            pltpu.sync_copy(rows0, acc.at[pl.ds(s * ROW_SLICE + r * K, K)])

        issue_idx(0, 0)
        wait_idx(0)
        issue_gather(0)
        issue_idx(1, 1)
        plsc.subcore_barrier()

        def body(t, _):
            wait_gather(0)
            wait_idx(1)
            issue_gather(1)
            sync_scatter(0)

            @pl.when(t + 1 < nb)
            def _():
                issue_idx(2 * (t + 1), 0)

            wait_gather(1)

            @pl.when(t + 1 < nb)
            def _():
                wait_idx(0)
                issue_gather(0)
            sync_scatter(1)

            @pl.when(t + 1 < nb)
            def _():
                issue_idx(2 * (t + 1) + 1, 1)
            return 0
        lax.fori_loop(0, nb, body, 0)

        plsc.subcore_barrier()
        for r in range(ROW_SLICE // K):
            row0 = s * ROW_SLICE + r * K
            pltpu.sync_copy(acc.at[pl.ds(row0, K)], rows0)
            pltpu.sync_copy(rows0, out_hbm.at[c, pl.ds(row0, K)])

    return pl.kernel(
        edge_kernel,
        out_type=jax.ShapeDtypeStruct((NC, N_PAD, dh), jnp.float32),
        mesh=_mesh(),
        scratch_types=[
            pltpu.VMEM((K,), jnp.int32),
            pltpu.VMEM((K,), jnp.int32),
            pltpu.VMEM((K,), jnp.int32),
            pltpu.VMEM((K,), jnp.int32),
            pltpu.VMEM((K, dh), jnp.float32),
            pltpu.VMEM((K, dh), jnp.float32),
            pltpu.VMEM_SHARED((N_PAD, dh), jnp.float32),
            pltpu.SemaphoreType.DMA,
            pltpu.SemaphoreType.DMA,
            pltpu.SemaphoreType.DMA,
            pltpu.SemaphoreType.DMA,
        ],
    )


def _tc_a0_body(x_ref, wlin_ref, w1_ref, h1_ref):
    xr = jnp.dot(x_ref[:], wlin_ref[:].T, preferred_element_type=jnp.float32)
    h1_ref[:] = jnp.dot(xr, w1_ref[:].T, preferred_element_type=jnp.float32)


def _tc_a1_body(h1_ref, degp_ref, table_ref, dinv_ref):
    deg = degp_ref[0, :] + degp_ref[1, :]
    dinv = lax.rsqrt(jnp.maximum(deg, 1e-12))
    h1s = h1_ref[:] * dinv[:, None]
    table_ref[0:N_PAD, :] = h1s[:, : D_H // 2]
    table_ref[N_PAD:, :] = h1s[:, D_H // 2:]
    dinv_ref[:] = dinv[:, None]


def _tc_c_body(raw_ref, dinv_ref, b1_ref, w2_ref, table_ref):
    dinv = dinv_ref[:]
    h = jnp.concatenate([raw_ref[0], raw_ref[1]], axis=1)
    h = jnp.maximum(h * dinv + b1_ref[:][None, :], 0.0)
    table_ref[:] = jnp.dot(h, w2_ref[:].T, preferred_element_type=jnp.float32) * dinv


def _tc_e_body(raw_ref, dinv_ref, b2_ref, z_ref):
    z = (raw_ref[0] + raw_ref[1]) * dinv_ref[:] + b2_ref[:][None, :]
    z_ref[:] = z[:N, :]


def kernel(x, train_pos_edge_index, Wlin, W1, b1, W2, b2):
    idt = train_pos_edge_index.dtype
    loop = jnp.arange(N, dtype=idt)
    pad = N + jnp.arange(E_PAD - E_TOT, dtype=idt) % (N_PAD - N)
    src = jnp.concatenate([train_pos_edge_index[0], loop, pad])
    dst = jnp.concatenate([train_pos_edge_index[1], loop, pad])
    src_both = jnp.concatenate([src, src + N_PAD])
    x_pad = jnp.pad(x, ((0, N_PAD - N), (0, 0)))

    deg_parts = _get_deg_kernel()(dst)

    h1u = pl.pallas_call(
        _tc_a0_body,
        out_shape=jax.ShapeDtypeStruct((N_PAD, D_H), jnp.float32),
    )(x_pad, Wlin, W1)

    table1, dinv = pl.pallas_call(
        _tc_a1_body,
        out_shape=(
            jax.ShapeDtypeStruct((2 * N_PAD, D_H // 2), jnp.float32),
            jax.ShapeDtypeStruct((N_PAD, 1), jnp.float32),
        ),
    )(h1u, deg_parts)

    raw1 = _make_edge_kernel("feat")(table1, src_both, dst)

    table2 = pl.pallas_call(
        _tc_c_body,
        out_shape=jax.ShapeDtypeStruct((N_PAD, D_OUT), jnp.float32),
    )(raw1, dinv, b1, W2)

    raw2 = _make_edge_kernel("edge")(table2, src_both, dst)

    z = pl.pallas_call(
        _tc_e_body,
        out_shape=jax.ShapeDtypeStruct((N, D_OUT), jnp.float32),
    )(raw2, dinv, b2)

    return z


# speedup vs baseline: 1.0045x; 1.0045x over previous
import functools

import jax
import jax.numpy as jnp
from jax import lax
from jax.experimental import pallas as pl
from jax.experimental.pallas import tpu as pltpu
from jax.experimental.pallas import tpu_sc as plsc

N = 10000
E = 320000
D_IN = 128
D_H = 256
D_OUT = 128

NC = 2
NS = 16
LANES = 16

N_PAD = 10240
ROW_SLICE = N_PAD // NS
E_TOT = E + N
K = 128
E_PAD = 8192 * 41
E_F = 4096 * 81
PT_L = E_F // NS
PT_D = E_PAD // (NC * NS)

@functools.cache
def _mesh():
    return plsc.VectorSubcoreMesh(
        core_axis_name="c", subcore_axis_name="s", num_cores=NC, num_subcores=NS
    )


def _zero_1d(buf, n):
    def w(i, _):
        buf[pl.ds(i * LANES, LANES)] = jnp.zeros((LANES,), jnp.float32)
        return 0
    lax.fori_loop(0, n // LANES, w, 0)


def _zero_2d(buf, rows, cols):
    def w(r, _):
        for j in range(cols // LANES):
            buf[r, pl.ds(j * LANES, LANES)] = jnp.zeros((LANES,), jnp.float32)
        return 0
    lax.fori_loop(0, rows, w, 0)


@functools.cache
def _get_deg_kernel():
    return pl.kernel(
        _deg_body,
        out_type=jax.ShapeDtypeStruct((NC, N_PAD), jnp.float32),
        mesh=_mesh(),
        scratch_types=[
            pltpu.VMEM((K,), jnp.int32),
            pltpu.VMEM((K,), jnp.int32),
            pltpu.VMEM((K,), jnp.float32),
            pltpu.VMEM((ROW_SLICE,), jnp.float32),
            pltpu.VMEM_SHARED((N_PAD,), jnp.float32),
            pltpu.SemaphoreType.DMA,
            pltpu.SemaphoreType.DMA,
        ],
    )


def _deg_body(dst_hbm, out_hbm, d0, d1, ones_v, buf_v, acc, isem0, isem1):
    c = lax.axis_index("c")
    s = lax.axis_index("s")
    idxs = (d0, d1)
    isems = (isem0, isem1)

    def w1(i, _):
        ones_v[pl.ds(i * LANES, LANES)] = jnp.full((LANES,), 1.0, jnp.float32)
        return 0
    lax.fori_loop(0, K // LANES, w1, 0)
    _zero_1d(buf_v, ROW_SLICE)
    pltpu.sync_copy(buf_v, acc.at[pl.ds(s * ROW_SLICE, ROW_SLICE)])

    base = c * (E_PAD // NC) + s * PT_D
    nb = (PT_D // K) // 2

    def issue_idx(g, p):
        pltpu.async_copy(dst_hbm.at[pl.ds(base + g * K, K)],
                         idxs[p], isems[p])

    def wait_idx(p):
        pltpu.make_async_copy(dst_hbm.at[pl.ds(0, K)],
                              idxs[p], isems[p]).wait()

    issue_idx(0, 0)
    plsc.subcore_barrier()

    def body(t, _):
        wait_idx(0)
        issue_idx(2 * t + 1, 1)
        pltpu.sync_copy(ones_v, acc.at[d0], add=True)
        wait_idx(1)

        @pl.when(t + 1 < nb)
        def _():
            issue_idx(2 * (t + 1), 0)
        pltpu.sync_copy(ones_v, acc.at[d1], add=True)
        return 0
    lax.fori_loop(0, nb, body, 0)

    plsc.subcore_barrier()
    pltpu.sync_copy(acc.at[pl.ds(s * ROW_SLICE, ROW_SLICE)], buf_v)
    pltpu.sync_copy(buf_v, out_hbm.at[c, pl.ds(s * ROW_SLICE, ROW_SLICE)])


@functools.cache
def _make_edge_kernel(mode):
    feat = mode == "feat"
    dh = 128
    nch = (PT_L if feat else PT_D) // K
    nb = nch // 2
    assert nb * 2 == nch

    def edge_kernel(table_hbm, srcb_hbm, dst_hbm, out_hbm,
                    src0, src1, dst0, dst1, rows0, rows1, acc,
                    gsem0, gsem1, isem0, isem1):
        c = lax.axis_index("c")
        s = lax.axis_index("s")
        srcs = (src0, src1)
        dsts = (dst0, dst1)
        rows = (rows0, rows1)
        gsems = (gsem0, gsem1)
        isems = (isem0, isem1)

        if feat:
            sbase = c * E_PAD + s * PT_L
            dbase = s * PT_L
        else:
            sbase = c * (E_PAD // NC) + s * PT_D
            dbase = sbase

        def issue_idx(g, p):
            pltpu.async_copy(srcb_hbm.at[pl.ds(sbase + g * K, K)],
                             srcs[p], isems[p])
            pltpu.async_copy(dst_hbm.at[pl.ds(dbase + g * K, K)],
                             dsts[p], isems[p])

        def wait_idx(p):
            pltpu.make_async_copy(srcb_hbm.at[pl.ds(0, K)],
                                  srcs[p], isems[p]).wait()
            pltpu.make_async_copy(dst_hbm.at[pl.ds(0, K)],
                                  dsts[p], isems[p]).wait()

        def issue_gather(p):
            pltpu.async_copy(table_hbm.at[srcs[p]], rows[p], gsems[p])

        def wait_gather(p):
            pltpu.make_async_copy(table_hbm.at[pl.ds(0, K)],
                                  rows[p], gsems[p]).wait()

        def sync_scatter(p):
            pltpu.sync_copy(rows[p], acc.at[dsts[p]], add=True)

        _zero_2d(rows0, K, dh)
        for r in range(ROW_SLICE // K):
            pltpu.sync_copy(rows0, acc.at[pl.ds(s * ROW_SLICE + r * K, K)])

        issue_idx(0, 0)
        wait_idx(0)
        issue_gather(0)
        issue_idx(1, 1)
        plsc.subcore_barrier()

        def body(t, _):
            wait_gather(0)
            wait_idx(1)
            issue_gather(1)
            sync_scatter(0)

            @pl.when(t + 1 < nb)
            def _():
                issue_idx(2 * (t + 1), 0)

            wait_gather(1)

            @pl.when(t + 1 < nb)
            def _():
                wait_idx(0)
                issue_gather(0)
            sync_scatter(1)

            @pl.when(t + 1 < nb)
            def _():
                issue_idx(2 * (t + 1) + 1, 1)
            return 0
        lax.fori_loop(0, nb, body, 0)

        plsc.subcore_barrier()
        for r in range(ROW_SLICE // K):
            row0 = s * ROW_SLICE + r * K
            pltpu.sync_copy(acc.at[pl.ds(row0, K)], rows0)
            pltpu.sync_copy(rows0, out_hbm.at[c, pl.ds(row0, K)])

    return pl.kernel(
        edge_kernel,
        out_type=jax.ShapeDtypeStruct((NC, N_PAD, dh), jnp.float32),
        mesh=_mesh(),
        scratch_types=[
            pltpu.VMEM((K,), jnp.int32),
            pltpu.VMEM((K,), jnp.int32),
            pltpu.VMEM((K,), jnp.int32),
            pltpu.VMEM((K,), jnp.int32),
            pltpu.VMEM((K, dh), jnp.float32),
            pltpu.VMEM((K, dh), jnp.float32),
            pltpu.VMEM_SHARED((N_PAD, dh), jnp.float32),
            pltpu.SemaphoreType.DMA,
            pltpu.SemaphoreType.DMA,
            pltpu.SemaphoreType.DMA,
            pltpu.SemaphoreType.DMA,
        ],
    )


def _tc_a_body(x_ref, wlin_ref, w1_ref, degp_ref, table_ref, dinv_ref):
    deg = degp_ref[0, :] + degp_ref[1, :]
    dinv = lax.rsqrt(jnp.maximum(deg, 1e-12))
    xr = jnp.dot(x_ref[:], wlin_ref[:].T, preferred_element_type=jnp.float32)
    h1 = jnp.dot(xr, w1_ref[:].T, preferred_element_type=jnp.float32)
    h1s = h1 * dinv[:, None]
    table_ref[0:N_PAD, :] = h1s[:, : D_H // 2]
    table_ref[N_PAD:, :] = h1s[:, D_H // 2:]
    dinv_ref[:] = dinv[:, None]


def _tc_c_body(raw_ref, dinv_ref, b1_ref, w2_ref, table_ref):
    dinv = dinv_ref[:]
    h = jnp.concatenate([raw_ref[0], raw_ref[1]], axis=1)
    h = jnp.maximum(h * dinv + b1_ref[:][None, :], 0.0)
    table_ref[:] = jnp.dot(h, w2_ref[:].T, preferred_element_type=jnp.float32) * dinv


def _tc_e_body(raw_ref, dinv_ref, b2_ref, z_ref):
    z = (raw_ref[0] + raw_ref[1]) * dinv_ref[:] + b2_ref[:][None, :]
    z_ref[:] = z[:N, :]


def kernel(x, train_pos_edge_index, Wlin, W1, b1, W2, b2):
    idt = train_pos_edge_index.dtype
    loop = jnp.arange(N, dtype=idt)
    pad = N + jnp.arange(E_PAD - E_TOT, dtype=idt) % (N_PAD - N)
    src = jnp.concatenate([train_pos_edge_index[0], loop, pad])
    dst = jnp.concatenate([train_pos_edge_index[1], loop, pad])
    src_both = jnp.concatenate([src, src + N_PAD])
    x_pad = jnp.pad(x, ((0, N_PAD - N), (0, 0)))

    deg_parts = _get_deg_kernel()(dst)

    table1, dinv = pl.pallas_call(
        _tc_a_body,
        out_shape=(
            jax.ShapeDtypeStruct((2 * N_PAD, D_H // 2), jnp.float32),
            jax.ShapeDtypeStruct((N_PAD, 1), jnp.float32),
        ),
    )(x_pad, Wlin, W1, deg_parts)

    raw1 = _make_edge_kernel("feat")(table1, src_both, dst)

    table2 = pl.pallas_call(
        _tc_c_body,
        out_shape=jax.ShapeDtypeStruct((N_PAD, D_OUT), jnp.float32),
    )(raw1, dinv, b1, W2)

    raw2 = _make_edge_kernel("edge")(table2, src_both, dst)

    z = pl.pallas_call(
        _tc_e_body,
        out_shape=jax.ShapeDtypeStruct((N, D_OUT), jnp.float32),
    )(raw2, dinv, b2)

    return z
